# Initial kernel scaffold; baseline (speedup 1.0000x reference)
import jax
import jax.numpy as jnp
from jax.experimental import pallas as pl


def kernel(x, router_scale, router_logits, gating_einsum, linear, per_expert_scale):
    raise NotImplementedError("write your pallas kernel here")


# speedup vs baseline: 11.9979x; 11.9979x over previous
import functools

import jax
import jax.numpy as jnp
from jax import lax
from jax.experimental import pallas as pl
from jax.experimental.pallas import tpu as pltpu

_L = 2048
_D = 768
_H = 64
_E = 64
_EP = 4
_STEPS = _E // _EP


def _moe_body(x_ref, rl_ref, rs_ref, ge_ref, lin_ref, pes_ref, out_ref, eid_ref):
    i = pl.program_id(0)
    x = x_ref[...]

    @pl.when(i == 0)
    def _route():
        x32 = x
        var = jnp.mean(x32 * x32, axis=1, keepdims=True)
        ri = x32 * lax.rsqrt(var + 1e-6)
        ri = ri * lax.rsqrt(jnp.float32(_D)) * rs_ref[...]
        logits = lax.dot_general(ri, rl_ref[...], (((1,), (0,)), ((), ())),
                                 preferred_element_type=jnp.float32)
        m = jnp.max(logits, axis=1, keepdims=True)
        ids = lax.broadcasted_iota(jnp.int32, (_L, _E), 1)
        cand = jnp.where(logits == m, ids, _E)
        eid_ref[...] = jnp.min(cand, axis=1, keepdims=True)
        out_ref[...] = jnp.zeros((_L, _D), jnp.float32)

    w0 = ge_ref[0, :, 0, :, :].reshape(_EP * _H, _D)
    w1 = ge_ref[0, :, 1, :, :].reshape(_EP * _H, _D)
    g0 = lax.dot_general(x, w0, (((1,), (1,)), ((), ())),
                         preferred_element_type=jnp.float32)
    g1 = lax.dot_general(x, w1, (((1,), (1,)), ((), ())),
                         preferred_element_type=jnp.float32)
    act = jax.nn.gelu(g0) * g1

    col_expert = lax.broadcasted_iota(jnp.int32, (_L, _EP * _H), 1) // _H + _EP * i
    scale = jnp.where(eid_ref[...] == col_expert, pes_ref[0], 0.0)
    act = act * scale

    out_ref[...] += lax.dot_general(act, lin_ref[0], (((1,), (0,)), ((), ())),
                                    preferred_element_type=jnp.float32)


@jax.jit
def kernel(x, router_scale, router_logits, gating_einsum, linear, per_expert_scale):
    B, L, D = x.shape
    x2 = x.reshape(L, D)
    rs = router_scale.reshape(1, D)
    ge = gating_einsum.reshape(_STEPS, _EP, 2, _H, D)
    lin = linear.reshape(_STEPS, _EP * _H, D)
    pes = jnp.repeat(per_expert_scale, _H).reshape(_STEPS, 1, _EP * _H)

    out = pl.pallas_call(
        _moe_body,
        grid=(_STEPS,),
        in_specs=[
            pl.BlockSpec((L, D), lambda i: (0, 0)),
            pl.BlockSpec((D, _E), lambda i: (0, 0)),
            pl.BlockSpec((1, D), lambda i: (0, 0)),
            pl.BlockSpec((1, _EP, 2, _H, D), lambda i: (i, 0, 0, 0, 0)),
            pl.BlockSpec((1, _EP * _H, D), lambda i: (i, 0, 0)),
            pl.BlockSpec((1, 1, _EP * _H), lambda i: (i, 0, 0)),
        ],
        out_specs=pl.BlockSpec((L, D), lambda i: (0, 0)),
        out_shape=jax.ShapeDtypeStruct((L, D), jnp.float32),
        scratch_shapes=[pltpu.VMEM((L, 1), jnp.int32)],
        compiler_params=pltpu.CompilerParams(
            dimension_semantics=("arbitrary",),
        ),
    )(x2, router_logits, rs, ge, lin, pes)
    return out.reshape(B, L, D)
